# Initial kernel scaffold; baseline (speedup 1.0000x reference)
import jax
import jax.numpy as jnp
from jax.experimental import pallas as pl


def kernel(x_batch, adj, k1, k2, Wd1, bd1, Wd2, bd2):
    raise NotImplementedError("write your pallas kernel here")


# speedup vs baseline: 5.8699x; 5.8699x over previous
import jax
import jax.numpy as jnp
from jax.experimental import pallas as pl


def _diffnet_kernel(x_ref, a_ref, k1t_ref, k2t_ref, w1_ref, b1_ref,
                    w2_ref, b2_ref, out_ref):
    x = x_ref[0]
    a = a_ref[0]

    s = jnp.sum(x, axis=1, keepdims=True)
    u0 = jnp.sum(s)
    u1 = jax.lax.dot(a, s, preferred_element_type=jnp.float32)
    u2 = jax.lax.dot(a * a, s, preferred_element_type=jnp.float32)

    th0 = k1t_ref[0:1, :]
    th1 = k1t_ref[1:2, :]
    th2 = k1t_ref[2:3, :]
    h1 = jnp.tanh(u2 * th0 + u1 * th1 + u0 * th2)

    t = jnp.sum(h1, axis=1, keepdims=True)
    tt = jnp.sum(t)
    v = jax.lax.dot(a, t, preferred_element_type=jnp.float32)

    ph0 = k2t_ref[0:1, :]
    ph1 = k2t_ref[1:2, :]
    h2 = jnp.tanh(v * ph0 + tt * ph1)

    pooled = jnp.sum(h2, axis=0, keepdims=True)
    d1 = jnp.tanh(
        jax.lax.dot(pooled, w1_ref[...], preferred_element_type=jnp.float32)
        + b1_ref[...])
    logits = (jax.lax.dot(d1, w2_ref[...], preferred_element_type=jnp.float32)
              + b2_ref[...])
    m = jnp.max(logits, axis=1, keepdims=True)
    e = jnp.exp(logits - m)
    out_ref[0] = e / jnp.sum(e, axis=1, keepdims=True)


def kernel(x_batch, adj, k1, k2, Wd1, bd1, Wd2, bd2):
    B, N, F = x_batch.shape
    C1 = k1.shape[0]
    C2 = k2.shape[0]
    k1t = k1.T
    k2t = k2.T
    b1 = bd1.reshape(1, -1)
    b2 = bd2.reshape(1, -1)

    out = pl.pallas_call(
        _diffnet_kernel,
        grid=(B,),
        in_specs=[
            pl.BlockSpec((1, N, F), lambda b: (b, 0, 0)),
            pl.BlockSpec((1, N, N), lambda b: (b, 0, 0)),
            pl.BlockSpec(k1t.shape, lambda b: (0, 0)),
            pl.BlockSpec(k2t.shape, lambda b: (0, 0)),
            pl.BlockSpec(Wd1.shape, lambda b: (0, 0)),
            pl.BlockSpec(b1.shape, lambda b: (0, 0)),
            pl.BlockSpec(Wd2.shape, lambda b: (0, 0)),
            pl.BlockSpec(b2.shape, lambda b: (0, 0)),
        ],
        out_specs=pl.BlockSpec((1, 1, 10), lambda b: (b, 0, 0)),
        out_shape=jax.ShapeDtypeStruct((B, 1, 10), jnp.float32),
    )(x_batch, adj, k1t, k2t, Wd1, b1, Wd2, b2)
    return out.reshape(B, 10)


# speedup vs baseline: 12.2770x; 2.0915x over previous
import jax
import jax.numpy as jnp
from jax.experimental import pallas as pl


def _diffnet_kernel(x_ref, a_ref, k1t_ref, k2t_ref, w1_ref, b1_ref,
                    w2_ref, b2_ref, out_ref):
    B = x_ref.shape[0]
    x = x_ref[...]
    a = a_ref[...]

    s = jnp.sum(x, axis=2, keepdims=True)
    u0 = jnp.sum(s, axis=(1, 2), keepdims=True)

    a2 = a * a
    u1 = jnp.stack([
        jax.lax.dot(a[b], s[b], preferred_element_type=jnp.float32)
        for b in range(B)])
    u2 = jnp.stack([
        jax.lax.dot(a2[b], s[b], preferred_element_type=jnp.float32)
        for b in range(B)])

    th0 = k1t_ref[0:1, :].reshape(1, 1, -1)
    th1 = k1t_ref[1:2, :].reshape(1, 1, -1)
    th2 = k1t_ref[2:3, :].reshape(1, 1, -1)
    h1 = jnp.tanh(u2 * th0 + u1 * th1 + u0 * th2)

    t = jnp.sum(h1, axis=2, keepdims=True)
    tt = jnp.sum(t, axis=(1, 2), keepdims=True)
    v = jnp.stack([
        jax.lax.dot(a[b], t[b], preferred_element_type=jnp.float32)
        for b in range(B)])

    ph0 = k2t_ref[0:1, :].reshape(1, 1, -1)
    ph1 = k2t_ref[1:2, :].reshape(1, 1, -1)
    h2 = jnp.tanh(v * ph0 + tt * ph1)

    pooled = jnp.sum(h2, axis=1)
    d1 = jnp.tanh(
        jax.lax.dot(pooled, w1_ref[...], preferred_element_type=jnp.float32)
        + b1_ref[...])
    logits = (jax.lax.dot(d1, w2_ref[...], preferred_element_type=jnp.float32)
              + b2_ref[...])
    m = jnp.max(logits, axis=1, keepdims=True)
    e = jnp.exp(logits - m)
    out_ref[...] = e / jnp.sum(e, axis=1, keepdims=True)


def kernel(x_batch, adj, k1, k2, Wd1, bd1, Wd2, bd2):
    B, N, F = x_batch.shape
    k1t = k1.T
    k2t = k2.T
    b1 = bd1.reshape(1, -1)
    b2 = bd2.reshape(1, -1)

    return pl.pallas_call(
        _diffnet_kernel,
        out_shape=jax.ShapeDtypeStruct((B, 10), jnp.float32),
    )(x_batch, adj, k1t, k2t, Wd1, b1, Wd2, b2)
